# Initial kernel scaffold; baseline (speedup 1.0000x reference)
import functools

import numpy as np
import jax
import jax.numpy as jnp
from jax import lax
from jax.experimental import pallas as pl
from jax.experimental.pallas import tpu as pltpu
from jax.experimental.pallas import tpu_sc as plsc


def _shuffle_indexes(num_patches_per_side, remain_T, rng):
    grid_size = num_patches_per_side ** 2
    mask_T = grid_size - remain_T
    block_side = int(mask_T ** 0.5)
    max_i = num_patches_per_side - block_side
    max_j = num_patches_per_side - block_side
    i = rng.randint(0, max_i + 1)
    j = rng.randint(0, max_j + 1)
    mask = np.zeros((num_patches_per_side, num_patches_per_side), dtype=np.float32)
    mask[i:i + block_side, j:j + block_side] = 1
    mask = mask.flatten()
    fwd = np.where(mask == 0)[0]
    bwd = np.argsort(np.concatenate((fwd, np.where(mask == 1)[0])))
    return fwd, bwd


@functools.lru_cache(maxsize=None)
def _build(T, B, C):
    ratio = 0.75
    remain_T = int(T * (1 - ratio))
    nps = int(T ** 0.5)
    assert T == nps ** 2
    rng = np.random.RandomState(0)
    idx_pairs = [_shuffle_indexes(nps, remain_T, rng) for _ in range(B)]
    fwd_np = np.stack([p[0] for p in idx_pairs], axis=-1).astype(np.int32)
    bwd_np = np.stack([p[1] for p in idx_pairs], axis=-1).astype(np.int32)

    flat_idx = (fwd_np[:remain_T].astype(np.int64) * B
                + np.arange(B, dtype=np.int64)[None, :]).reshape(-1).astype(np.int32)

    info = plsc.get_sparse_core_info()
    NW = info.num_cores * info.num_subcores
    R = remain_T * B
    assert R % NW == 0
    r_per_w = R // NW
    CHUNK = 128
    assert r_per_w % CHUNK == 0
    n_chunks = r_per_w // CHUNK
    idx_arr = jnp.asarray(flat_idx.reshape(NW, n_chunks, CHUNK))

    mesh = plsc.VectorSubcoreMesh(core_axis_name="c", subcore_axis_name="s")

    @functools.partial(
        pl.kernel,
        mesh=mesh,
        out_type=jax.ShapeDtypeStruct((R, C), jnp.float32),
        scratch_types=[
            pltpu.VMEM((n_chunks, CHUNK), jnp.int32),
            pltpu.VMEM((CHUNK, C), jnp.float32),
            pltpu.VMEM((CHUNK, C), jnp.float32),
            pltpu.SemaphoreType.DMA,
            pltpu.SemaphoreType.DMA,
        ],
    )
    def gather_rows(table_hbm, idx_hbm, out_hbm, idx_v, rows0, rows1, sem0, sem1):
        wid = lax.axis_index("s") * info.num_cores + lax.axis_index("c")
        base = wid * r_per_w
        pltpu.sync_copy(idx_hbm.at[wid], idx_v)
        bufs = (rows0, rows1)
        sems = (sem0, sem1)
        copies = [None, None]
        copies[0] = pltpu.async_copy(table_hbm.at[idx_v.at[0]], bufs[0], sems[0])
        for j in range(n_chunks):
            if j + 1 < n_chunks:
                copies[(j + 1) % 2] = pltpu.async_copy(
                    table_hbm.at[idx_v.at[j + 1]], bufs[(j + 1) % 2], sems[(j + 1) % 2])
            copies[j % 2].wait()
            pltpu.sync_copy(bufs[j % 2], out_hbm.at[pl.ds(base + j * CHUNK, CHUNK)])

    fwd_j = jnp.asarray(fwd_np)
    bwd_j = jnp.asarray(bwd_np)
    return gather_rows, idx_arr, fwd_j, bwd_j, remain_T


def kernel(patches):
    T, B, C = patches.shape
    gather_rows, idx_arr, fwd, bwd, remain_T = _build(T, B, C)
    table = patches.reshape(T * B, C)
    out = gather_rows(table, idx_arr)
    return out.reshape(remain_T, B, C), fwd, bwd


# speedup vs baseline: 1.4488x; 1.4488x over previous
import functools

import numpy as np
import jax
import jax.numpy as jnp
from jax import lax
from jax.experimental import pallas as pl
from jax.experimental.pallas import tpu as pltpu
from jax.experimental.pallas import tpu_sc as plsc


def _shuffle_indexes(num_patches_per_side, remain_T, rng):
    grid_size = num_patches_per_side ** 2
    mask_T = grid_size - remain_T
    block_side = int(mask_T ** 0.5)
    max_i = num_patches_per_side - block_side
    max_j = num_patches_per_side - block_side
    i = rng.randint(0, max_i + 1)
    j = rng.randint(0, max_j + 1)
    mask = np.zeros((num_patches_per_side, num_patches_per_side), dtype=np.float32)
    mask[i:i + block_side, j:j + block_side] = 1
    mask = mask.flatten()
    fwd = np.where(mask == 0)[0]
    bwd = np.argsort(np.concatenate((fwd, np.where(mask == 1)[0])))
    return fwd, bwd


@functools.lru_cache(maxsize=None)
def _build(T, B, C):
    ratio = 0.75
    remain_T = int(T * (1 - ratio))
    nps = int(T ** 0.5)
    assert T == nps ** 2
    rng = np.random.RandomState(0)
    idx_pairs = [_shuffle_indexes(nps, remain_T, rng) for _ in range(B)]
    fwd_np = np.stack([p[0] for p in idx_pairs], axis=-1).astype(np.int32)
    bwd_np = np.stack([p[1] for p in idx_pairs], axis=-1).astype(np.int32)

    flat_idx = (fwd_np[:remain_T].astype(np.int64) * B
                + np.arange(B, dtype=np.int64)[None, :]).reshape(-1).astype(np.int32)

    info = plsc.get_sparse_core_info()
    NW = info.num_cores * info.num_subcores
    R = remain_T * B
    assert R % NW == 0
    r_per_w = R // NW
    CHUNK = 128
    assert r_per_w % CHUNK == 0
    n_chunks = r_per_w // CHUNK
    idx_arr = jnp.asarray(flat_idx.reshape(NW, n_chunks, CHUNK))

    mesh = plsc.VectorSubcoreMesh(core_axis_name="c", subcore_axis_name="s")

    @functools.partial(
        pl.kernel,
        mesh=mesh,
        compiler_params=pltpu.CompilerParams(use_tc_tiling_on_sc=False),
        out_type=jax.ShapeDtypeStruct((R, C), jnp.float32),
        scratch_types=[
            pltpu.VMEM((n_chunks, CHUNK), jnp.int32),
            pltpu.VMEM((CHUNK, C), jnp.float32),
            pltpu.VMEM((CHUNK, C), jnp.float32),
            pltpu.SemaphoreType.DMA,
            pltpu.SemaphoreType.DMA,
        ],
    )
    def gather_rows(table_hbm, idx_hbm, out_hbm, idx_v, rows0, rows1, sem0, sem1):
        wid = lax.axis_index("s") * info.num_cores + lax.axis_index("c")
        base = wid * r_per_w
        pltpu.sync_copy(idx_hbm.at[wid], idx_v)
        bufs = (rows0, rows1)
        sems = (sem0, sem1)
        copies = [None, None]
        copies[0] = pltpu.async_copy(table_hbm.at[idx_v.at[0]], bufs[0], sems[0])
        for j in range(n_chunks):
            if j + 1 < n_chunks:
                copies[(j + 1) % 2] = pltpu.async_copy(
                    table_hbm.at[idx_v.at[j + 1]], bufs[(j + 1) % 2], sems[(j + 1) % 2])
            copies[j % 2].wait()
            pltpu.sync_copy(bufs[j % 2], out_hbm.at[pl.ds(base + j * CHUNK, CHUNK)])

    fwd_j = jnp.asarray(fwd_np)
    bwd_j = jnp.asarray(bwd_np)
    return gather_rows, idx_arr, fwd_j, bwd_j, remain_T


def kernel(patches):
    T, B, C = patches.shape
    gather_rows, idx_arr, fwd, bwd, remain_T = _build(T, B, C)
    table = patches.reshape(T * B, C)
    out = gather_rows(table, idx_arr)
    return out.reshape(remain_T, B, C), fwd, bwd


# speedup vs baseline: 2.0535x; 1.4174x over previous
import functools

import numpy as np
import jax
import jax.numpy as jnp
from jax import lax
from jax.experimental import pallas as pl
from jax.experimental.pallas import tpu as pltpu
from jax.experimental.pallas import tpu_sc as plsc


def _shuffle_indexes(num_patches_per_side, remain_T, rng):
    grid_size = num_patches_per_side ** 2
    mask_T = grid_size - remain_T
    block_side = int(mask_T ** 0.5)
    max_i = num_patches_per_side - block_side
    max_j = num_patches_per_side - block_side
    i = rng.randint(0, max_i + 1)
    j = rng.randint(0, max_j + 1)
    mask = np.zeros((num_patches_per_side, num_patches_per_side), dtype=np.float32)
    mask[i:i + block_side, j:j + block_side] = 1
    mask = mask.flatten()
    fwd = np.where(mask == 0)[0]
    bwd = np.argsort(np.concatenate((fwd, np.where(mask == 1)[0])))
    return fwd, bwd


@functools.lru_cache(maxsize=None)
def _build(T, B, C):
    ratio = 0.75
    remain_T = int(T * (1 - ratio))
    nps = int(T ** 0.5)
    assert T == nps ** 2
    rng = np.random.RandomState(0)
    idx_pairs = [_shuffle_indexes(nps, remain_T, rng) for _ in range(B)]
    fwd_np = np.stack([p[0] for p in idx_pairs], axis=-1).astype(np.int32)
    bwd_np = np.stack([p[1] for p in idx_pairs], axis=-1).astype(np.int32)

    idx_flat = fwd_np[:remain_T].reshape(remain_T * B).astype(np.int32)
    idx_arr = jnp.asarray(idx_flat)

    info = plsc.get_sparse_core_info()
    NW = info.num_cores * info.num_subcores
    assert remain_T % NW == 0
    t_per_w = remain_T // NW
    LANES = 16
    assert B % LANES == 0
    nblk = B // LANES

    mesh = plsc.VectorSubcoreMesh(core_axis_name="c", subcore_axis_name="s")

    @functools.partial(
        pl.kernel,
        mesh=mesh,
        out_type=jax.ShapeDtypeStruct((remain_T, B, C), jnp.float32),
        scratch_types=[
            pltpu.VMEM((t_per_w * B,), jnp.int32),
            pltpu.VMEM((B, C), jnp.float32),
            pltpu.VMEM((B, C), jnp.float32),
            pltpu.SemaphoreType.DMA,
            pltpu.SemaphoreType.DMA,
            pltpu.SemaphoreType.DMA,
            pltpu.SemaphoreType.DMA,
        ],
    )
    def gather_rows(in_hbm, idx_hbm, out_hbm, idx_v, v0, v1, gs0, gs1, os0, os1):
        wid = lax.axis_index("s") * info.num_cores + lax.axis_index("c")
        tbase = wid * t_per_w
        pltpu.sync_copy(idx_hbm.at[pl.ds(tbase * B, t_per_w * B)], idx_v)
        vbufs = (v0, v1)
        gsems = (gs0, gs1)
        osems = (os0, os1)
        ocopies = [None, None]

        def issue_slab(ti):
            vb, gs = vbufs[ti % 2], gsems[ti % 2]
            for blk in range(nblk):
                vec = idx_v[pl.ds(ti * B + blk * LANES, LANES)]
                for lane in range(LANES):
                    src = vec[lane]
                    b = blk * LANES + lane
                    pltpu.async_copy(in_hbm.at[src, b], vb.at[b], gs)

        def drain_slab(p):
            pltpu.make_async_copy(in_hbm.at[0], vbufs[p], gsems[p]).wait()

        issue_slab(0)
        for ti in range(t_per_w):
            p = ti % 2
            if ti + 1 < t_per_w:
                if ocopies[p ^ 1] is not None:
                    ocopies[p ^ 1].wait()
                issue_slab(ti + 1)
            drain_slab(p)
            ocopies[p] = pltpu.async_copy(
                vbufs[p], out_hbm.at[tbase + ti], osems[p])
        for p in range(2):
            if ocopies[p] is not None:
                ocopies[p].wait()

    fwd_j = jnp.asarray(fwd_np)
    bwd_j = jnp.asarray(bwd_np)
    return gather_rows, idx_arr, fwd_j, bwd_j


def kernel(patches):
    T, B, C = patches.shape
    gather_rows, idx_arr, fwd, bwd = _build(T, B, C)
    out = gather_rows(patches, idx_arr)
    return out, fwd, bwd
